# Initial kernel scaffold; baseline (speedup 1.0000x reference)
import jax
import jax.numpy as jnp
from jax.experimental import pallas as pl


def kernel(query, memory):
    raise NotImplementedError("write your pallas kernel here")


# speedup vs baseline: 5.0596x; 5.0596x over previous
import jax
import jax.numpy as jnp
from jax.experimental import pallas as pl
from jax.experimental.pallas import tpu as pltpu

NEIGH = 10
EPS = 1e-5
BK = 2048
LANES = 128


def _body(q_ref, m_ref, out_ref, r_ref, sum_ref, *, k_valid, grid):
    i = pl.program_id(0)
    q_rows = q_ref.shape[0]
    w = NEIGH * LANES

    @pl.when(i == 0)
    def _init():
        r_ref[...] = jnp.full((q_rows, w), -1.0, jnp.float32)
        sum_ref[0] = 0.0

    q = q_ref[...]
    m = m_ref[...]
    dot = jax.lax.dot_general(
        q, m, (((1,), (1,)), ((), ())),
        preferred_element_type=jnp.float32,
        precision=jax.lax.Precision.HIGHEST)
    q2 = jnp.sum(q * q, axis=1, keepdims=True)
    m2 = jnp.sum(m * m, axis=1)
    d2 = jnp.maximum(q2 + m2[None, :] - 2.0 * dot, 0.0)
    col = i * BK + jax.lax.broadcasted_iota(jnp.int32, (1, BK), 1)
    valid = col < k_valid
    sum_ref[0] += jnp.sum(jnp.where(valid, d2, 0.0))
    d2m = jnp.where(valid, d2, -1.0)

    slots = [r_ref[:, j * LANES:(j + 1) * LANES] for j in range(NEIGH)]
    for c in range(BK // LANES):
        cur = d2m[:, c * LANES:(c + 1) * LANES]
        for j in range(NEIGH):
            hi = jnp.maximum(slots[j], cur)
            cur = jnp.minimum(slots[j], cur)
            slots[j] = hi
    for j in range(NEIGH):
        r_ref[:, j * LANES:(j + 1) * LANES] = slots[j]

    @pl.when(i == grid - 1)
    def _final():
        cand = r_ref[...]
        mean = sum_ref[0] / (q_rows * k_valid)
        ii = jax.lax.broadcasted_iota(jnp.int32, (q_rows, w), 1)
        s = jnp.zeros((q_rows, 1), jnp.float32)
        for _ in range(NEIGH):
            mx = jnp.max(cand, axis=1, keepdims=True)
            loc = jnp.min(jnp.where(cand == mx, ii, 1 << 30), axis=1,
                          keepdims=True)
            cand = jnp.where(ii == loc, -2.0, cand)
            s = s + EPS / (mx / mean + EPS)
        out_ref[...] = 1.0 / jnp.sqrt(s + EPS)


def kernel(query, memory):
    q_rows, d = query.shape
    k_valid = memory.shape[0]
    grid = pl.cdiv(k_valid, BK)

    import functools
    body = functools.partial(_body, k_valid=k_valid, grid=grid)
    out = pl.pallas_call(
        body,
        grid=(grid,),
        in_specs=[
            pl.BlockSpec((q_rows, d), lambda i: (0, 0)),
            pl.BlockSpec((BK, d), lambda i: (i, 0)),
        ],
        out_specs=pl.BlockSpec((q_rows, 1), lambda i: (0, 0)),
        out_shape=jax.ShapeDtypeStruct((q_rows, 1), jnp.float32),
        scratch_shapes=[
            pltpu.VMEM((q_rows, NEIGH * LANES), jnp.float32),
            pltpu.SMEM((1,), jnp.float32),
        ],
        compiler_params=pltpu.CompilerParams(
            dimension_semantics=("arbitrary",)),
    )(query, memory)
    return out[:, 0]


# speedup vs baseline: 5.6109x; 1.1090x over previous
import jax
import jax.numpy as jnp
from jax.experimental import pallas as pl
from jax.experimental.pallas import tpu as pltpu

NEIGH = 10
EPS = 1e-5
BK = 2048
LANES = 128


def _batcher_pairs(n):
    pairs = []
    p = 1
    while p < n:
        k = p
        while k >= 1:
            for j in range(k % p, n - k, 2 * k):
                for i in range(0, min(k, n - j - k)):
                    if (i + j) // (2 * p) == (i + j + k) // (2 * p):
                        pairs.append((i + j, i + j + k))
            k //= 2
        p *= 2
    return pairs


def _prune(pairs, needed):
    needed = set(needed)
    kept = []
    for a, b in reversed(pairs):
        if a in needed or b in needed:
            kept.append((a, b))
            needed.add(a)
            needed.add(b)
    return list(reversed(kept))


_SORT16 = _prune(_batcher_pairs(16), range(NEIGH))
_B5 = _batcher_pairs(5)
_CLEAN10 = ([(i, i + 5) for i in range(5)] + _B5
            + [(a + 5, b + 5) for a, b in _B5])


def _body(q_ref, m_ref, out_ref, r_ref, sum_ref, *, k_valid, grid):
    i = pl.program_id(0)
    q_rows = q_ref.shape[0]
    w = NEIGH * LANES

    @pl.when(i == 0)
    def _init():
        r_ref[...] = jnp.full((q_rows, w), -1.0, jnp.float32)
        sum_ref[0] = 0.0

    q = q_ref[...]
    m = m_ref[...]
    dot = jax.lax.dot_general(
        q, m, (((1,), (1,)), ((), ())),
        preferred_element_type=jnp.float32,
        precision=jax.lax.Precision.HIGHEST)
    q2 = jnp.sum(q * q, axis=1, keepdims=True)
    m2 = jnp.sum(m * m, axis=1)
    d2 = jnp.maximum(q2 + m2[None, :] - 2.0 * dot, 0.0)
    col = i * BK + jax.lax.broadcasted_iota(jnp.int32, (1, BK), 1)
    valid = col < k_valid
    sum_ref[0] += jnp.sum(jnp.where(valid, d2, 0.0))
    d2m = jnp.where(valid, d2, -1.0)

    s = [d2m[:, c * LANES:(c + 1) * LANES] for c in range(BK // LANES)]
    for a, b in _SORT16:
        hi = jnp.maximum(s[a], s[b])
        lo = jnp.minimum(s[a], s[b])
        s[a], s[b] = hi, lo
    slots = [r_ref[:, j * LANES:(j + 1) * LANES] for j in range(NEIGH)]
    t = [jnp.maximum(slots[j], s[NEIGH - 1 - j]) for j in range(NEIGH)]
    for a, b in _CLEAN10:
        hi = jnp.maximum(t[a], t[b])
        lo = jnp.minimum(t[a], t[b])
        t[a], t[b] = hi, lo
    for j in range(NEIGH):
        r_ref[:, j * LANES:(j + 1) * LANES] = t[j]

    @pl.when(i == grid - 1)
    def _final():
        cand = r_ref[...]
        mean = sum_ref[0] / (q_rows * k_valid)
        ii = jax.lax.broadcasted_iota(jnp.int32, (q_rows, w), 1)
        s = jnp.zeros((q_rows, 1), jnp.float32)
        for _ in range(NEIGH):
            mx = jnp.max(cand, axis=1, keepdims=True)
            loc = jnp.min(jnp.where(cand == mx, ii, 1 << 30), axis=1,
                          keepdims=True)
            cand = jnp.where(ii == loc, -2.0, cand)
            s = s + EPS / (mx / mean + EPS)
        out_ref[...] = 1.0 / jnp.sqrt(s + EPS)


def kernel(query, memory):
    q_rows, d = query.shape
    k_valid = memory.shape[0]
    grid = pl.cdiv(k_valid, BK)

    import functools
    body = functools.partial(_body, k_valid=k_valid, grid=grid)
    out = pl.pallas_call(
        body,
        grid=(grid,),
        in_specs=[
            pl.BlockSpec((q_rows, d), lambda i: (0, 0)),
            pl.BlockSpec((BK, d), lambda i: (i, 0)),
        ],
        out_specs=pl.BlockSpec((q_rows, 1), lambda i: (0, 0)),
        out_shape=jax.ShapeDtypeStruct((q_rows, 1), jnp.float32),
        scratch_shapes=[
            pltpu.VMEM((q_rows, NEIGH * LANES), jnp.float32),
            pltpu.SMEM((1,), jnp.float32),
        ],
        compiler_params=pltpu.CompilerParams(
            dimension_semantics=("arbitrary",)),
    )(query, memory)
    return out[:, 0]
